# Initial kernel scaffold; baseline (speedup 1.0000x reference)
import jax
import jax.numpy as jnp
from jax.experimental import pallas as pl


def kernel(x_protein, hyperedge_protein_index, x_meta, hyperedge_meta_index, pc1_w1, pc1_b1, pc1_g, pc1_be, pc1_rm, pc1_rv, pc1_w2, pc1_b2, pc2_w1, pc2_b1, pc2_g, pc2_be, pc2_rm, pc2_rv, pc2_w2, pc2_b2, mc1_w1, mc1_b1, mc1_g, mc1_be, mc1_rm, mc1_rv, mc1_w2, mc1_b2, mc2_w1, mc2_b1, mc2_g, mc2_be, mc2_rm, mc2_rv, mc2_w2, mc2_b2, p12, m13):
    raise NotImplementedError("write your pallas kernel here")


# speedup vs baseline: 8.2005x; 8.2005x over previous
import functools

import jax
import jax.numpy as jnp
from jax import lax
from jax.experimental import pallas as pl
from jax.experimental.pallas import tpu as pltpu
from jax.experimental.pallas import tpu_sc as plsc

N = 10000
E = 320000
D = 128
CH = 128
NCH = E // CH
NSUB = 16
NCORE = 2
NPAD = 10240
RPT = NPAD // NSUB
ZR = 128


@functools.lru_cache(maxsize=None)
def _sc_mesh():
    return plsc.VectorSubcoreMesh(
        core_axis_name="c", subcore_axis_name="s",
        num_cores=NCORE, num_subcores=NSUB,
    )


def _n_chunks_for(s):
    return (NCH // NSUB) + (s < (NCH % NSUB)).astype(jnp.int32)


def _sc_hist_body(src4, dst4, ones_hbm, dv_out, bv_out, acc_d, acc_b, iv, jv,
                  ones_v, zer_v):
    c = lax.axis_index("c")
    s = lax.axis_index("s")
    nj = _n_chunks_for(s)
    pltpu.sync_copy(ones_hbm, ones_v)

    def _z(i, carry):
        zer_v[pl.ds(i * 16, 16)] = jnp.zeros((16,), jnp.float32)
        return carry
    lax.fori_loop(0, RPT // 16, _z, 0)

    for p in range(2):
        h = 2 * p + c
        pltpu.sync_copy(zer_v, acc_d.at[pl.ds(s * RPT, RPT)])
        pltpu.sync_copy(zer_v, acc_b.at[pl.ds(s * RPT, RPT)])
        plsc.subcore_barrier()

        def _body(j, carry):
            ch = s + NSUB * j
            pltpu.sync_copy(src4.at[h, ch, 0], iv)
            pltpu.sync_copy(dst4.at[h, ch, 0], jv)
            pltpu.sync_copy(ones_v, acc_d.at[iv], add=True)
            pltpu.sync_copy(ones_v, acc_b.at[jv], add=True)
            return carry

        lax.fori_loop(0, nj, _body, 0)
        plsc.subcore_barrier()
        pltpu.sync_copy(acc_d.at[pl.ds(s * RPT, RPT)],
                        dv_out.at[pl.ds(h * NPAD + s * RPT, RPT)])
        pltpu.sync_copy(acc_b.at[pl.ds(s * RPT, RPT)],
                        bv_out.at[pl.ds(h * NPAD + s * RPT, RPT)])
        plsc.subcore_barrier()


@functools.lru_cache(maxsize=None)
def _sc_hist_fn():
    return pl.kernel(
        _sc_hist_body,
        out_type=(
            jax.ShapeDtypeStruct((4 * NPAD,), jnp.float32),
            jax.ShapeDtypeStruct((4 * NPAD,), jnp.float32),
        ),
        mesh=_sc_mesh(),
        scratch_types=[
            pltpu.VMEM_SHARED((NPAD,), jnp.float32),
            pltpu.VMEM_SHARED((NPAD,), jnp.float32),
            pltpu.VMEM((CH,), jnp.int32),
            pltpu.VMEM((CH,), jnp.int32),
            pltpu.VMEM((CH,), jnp.float32),
            pltpu.VMEM((RPT,), jnp.float32),
        ],
    )


def _sc_hist(src4, dst4, ones_ch):
    dv, bv = _sc_hist_fn()(src4, dst4, ones_ch)
    return dv.reshape(4, NPAD), bv.reshape(4, NPAD)


def _sc_stage_body(xin, gidx, sidx, out, acc, gv, sv, rows, zer_v, sem):
    c = lax.axis_index("c")
    s = lax.axis_index("s")
    nj = _n_chunks_for(s)

    def _z(i, carry):
        r = i // (D // 16)
        k = (i % (D // 16)) * 16
        zer_v[r, pl.ds(k, 16)] = jnp.zeros((16,), jnp.float32)
        return carry
    lax.fori_loop(0, ZR * (D // 16), _z, 0)

    for p in range(2):
        h = 2 * p + c
        for q in range(RPT // ZR):
            pltpu.sync_copy(zer_v, acc.at[pl.ds(s * RPT + q * ZR, ZR)])
        plsc.subcore_barrier()

        def _body(j, carry):
            ch = s + NSUB * j
            pltpu.sync_copy(gidx.at[h, ch, 0], gv)
            pltpu.sync_copy(sidx.at[h, ch, 0], sv)
            pltpu.async_copy(xin.at[gv], rows, sem).wait()
            pltpu.sync_copy(rows, acc.at[sv], add=True)
            return carry

        lax.fori_loop(0, nj, _body, 0)
        plsc.subcore_barrier()
        pltpu.sync_copy(
            acc.at[pl.ds(s * RPT, RPT)],
            out.at[pl.ds(h * NPAD + s * RPT, RPT)],
        )
        plsc.subcore_barrier()


@functools.lru_cache(maxsize=None)
def _sc_stage_fn():
    return pl.kernel(
        _sc_stage_body,
        out_type=jax.ShapeDtypeStruct((4 * NPAD, D), jnp.float32),
        mesh=_sc_mesh(),
        scratch_types=[
            pltpu.VMEM_SHARED((NPAD, D), jnp.float32),
            pltpu.VMEM((CH,), jnp.int32),
            pltpu.VMEM((CH,), jnp.int32),
            pltpu.VMEM((CH, D), jnp.float32),
            pltpu.VMEM((ZR, D), jnp.float32),
            pltpu.SemaphoreType.DMA,
        ],
    )


def _sc_stage(xin_flat, gidx_abs, sidx_loc):
    return _sc_stage_fn()(xin_flat, gidx_abs, sidx_loc)


def _tc_scale_body(a_ref, bv_ref, o_ref):
    bv = bv_ref[0, 0, :]
    inv = jnp.where(bv > 0, 1.0 / bv, 0.0)
    o_ref[...] = a_ref[...] * inv[:, None]


def _tc_scale(a_flat, bv):
    return pl.pallas_call(
        _tc_scale_body,
        grid=(4,),
        in_specs=[
            pl.BlockSpec((NPAD, D), lambda i: (i, 0)),
            pl.BlockSpec((1, 1, NPAD), lambda i: (i, 0, 0)),
        ],
        out_specs=pl.BlockSpec((NPAD, D), lambda i: (i, 0)),
        out_shape=jax.ShapeDtypeStruct((4 * NPAD, D), jnp.float32),
    )(a_flat, bv.reshape(4, 1, NPAD))


def _tc_mid_body(c_ref, dv_ref, w1_ref, b1_ref, g_ref, be_ref, rm_ref, rv_ref,
                 w2_ref, o_ref):
    dv = dv_ref[0, 0, :]
    dinv = jnp.where(dv > 0, 1.0 / dv, 0.0)
    sg = g_ref[0, 0, :] / jnp.sqrt(rv_ref[0, 0, :] + 1e-5)
    t = be_ref[0, 0, :] - rm_ref[0, 0, :] * sg
    w1 = w1_ref[0]
    w2 = w2_ref[0]
    w1s = w1 * sg[:, None]
    weff = lax.dot_general(w1s, w2, (((0,), (1,)), ((), ())),
                           preferred_element_type=jnp.float32)
    bvec = (b1_ref[0, 0, :] * sg + t)[None, :]
    beff = lax.dot_general(bvec, w2, (((1,), (1,)), ((), ())),
                           preferred_element_type=jnp.float32)
    cd = c_ref[...] * dinv[:, None]
    o_ref[...] = lax.dot_general(cd, weff, (((1,), (0,)), ((), ())),
                                 preferred_element_type=jnp.float32) + beff


def _tc_mid(c_flat, dv, w1s_, b1s_, gs_, bes_, rms_, rvs_, w2s_):
    vec = lambda x: x.reshape(4, 1, 2 * D)
    return pl.pallas_call(
        _tc_mid_body,
        grid=(4,),
        in_specs=[
            pl.BlockSpec((NPAD, D), lambda i: (i, 0)),
            pl.BlockSpec((1, 1, NPAD), lambda i: (i, 0, 0)),
            pl.BlockSpec((1, 2 * D, D), lambda i: (i, 0, 0)),
            pl.BlockSpec((1, 1, 2 * D), lambda i: (i, 0, 0)),
            pl.BlockSpec((1, 1, 2 * D), lambda i: (i, 0, 0)),
            pl.BlockSpec((1, 1, 2 * D), lambda i: (i, 0, 0)),
            pl.BlockSpec((1, 1, 2 * D), lambda i: (i, 0, 0)),
            pl.BlockSpec((1, 1, 2 * D), lambda i: (i, 0, 0)),
            pl.BlockSpec((1, D, 2 * D), lambda i: (i, 0, 0)),
        ],
        out_specs=pl.BlockSpec((NPAD, D), lambda i: (i, 0)),
        out_shape=jax.ShapeDtypeStruct((4 * NPAD, D), jnp.float32),
    )(c_flat, dv.reshape(4, 1, NPAD), w1s_, vec(b1s_), vec(gs_), vec(bes_),
      vec(rms_), vec(rvs_), w2s_)


def _tc_final_body(c_ref, dv_ref, b2_ref, o_ref):
    dv = dv_ref[0, 0, :]
    dinv = jnp.where(dv > 0, 1.0 / dv, 0.0)
    o_ref[...] = c_ref[...] * dinv[:, None] + b2_ref[0, 0, :][None, :]


def _tc_final(c_flat, dv, b2s_):
    return pl.pallas_call(
        _tc_final_body,
        grid=(4,),
        in_specs=[
            pl.BlockSpec((NPAD, D), lambda i: (i, 0)),
            pl.BlockSpec((1, 1, NPAD), lambda i: (i, 0, 0)),
            pl.BlockSpec((1, 1, D), lambda i: (i, 0, 0)),
        ],
        out_specs=pl.BlockSpec((NPAD, D), lambda i: (i, 0)),
        out_shape=jax.ShapeDtypeStruct((4 * NPAD, D), jnp.float32),
    )(c_flat, dv.reshape(4, 1, NPAD), b2s_.reshape(4, 1, D))


_BR = 1024
_NBLK = NPAD // _BR


def _tc_loss_body(z1_ref, z2_ref, o_ref, sr, sc_, acc):
    i = pl.program_id(0)
    j = pl.program_id(1)

    @pl.when(jnp.logical_and(i == 0, j == 0))
    def _():
        acc[0] = 0.0
        acc[1] = 0.0
        acc[2] = 0.0

    z1 = z1_ref[...]
    z2 = z2_ref[...]
    n1 = z1 / jnp.maximum(
        jnp.sqrt(jnp.sum(z1 * z1, axis=1, keepdims=True)), 1e-12)
    n2 = z2 / jnp.maximum(
        jnp.sqrt(jnp.sum(z2 * z2, axis=1, keepdims=True)), 1e-12)
    s_blk = lax.dot_general(n1, n2, (((1,), (1,)), ((), ())),
                            preferred_element_type=jnp.float32) * (1.0 / 0.7)
    ri = lax.broadcasted_iota(jnp.int32, (_BR, _BR), 0)
    ci = lax.broadcasted_iota(jnp.int32, (_BR, _BR), 1)
    valid = jnp.logical_and(i * _BR + ri < N, j * _BR + ci < N)
    p_blk = jnp.where(valid, jnp.exp(s_blk), 0.0)

    @pl.when(j == 0)
    def _():
        sr[...] = jnp.zeros_like(sr)

    sr[...] = sr[...] + jnp.sum(p_blk, axis=1, keepdims=True)

    @pl.when(i == 0)
    def _():
        sc_[0, pl.ds(j * _BR, _BR)] = jnp.zeros((_BR,), jnp.float32)

    cur = sc_[0, pl.ds(j * _BR, _BR)]
    sc_[0, pl.ds(j * _BR, _BR)] = cur + jnp.sum(p_blk, axis=0)

    @pl.when(i == j)
    def _():
        dmask = jnp.logical_and(ri == ci, valid)
        acc[2] = acc[2] + jnp.sum(jnp.where(dmask, s_blk, 0.0))

    @pl.when(j == _NBLK - 1)
    def _():
        rv_ = lax.broadcasted_iota(jnp.int32, (_BR, 1), 0) + i * _BR < N
        acc[0] = acc[0] + jnp.sum(jnp.where(rv_, jnp.log(sr[...]), 0.0))

    @pl.when(jnp.logical_and(i == _NBLK - 1, j == _NBLK - 1))
    def _():
        cv_ = lax.broadcasted_iota(jnp.int32, (1, NPAD), 1) < N
        acc[1] = jnp.sum(jnp.where(cv_, jnp.log(sc_[...]), 0.0))
        val = 0.5 * ((acc[0] - acc[2]) + (acc[1] - acc[2])) / N
        o_ref[...] = jnp.full((1, 1), val, jnp.float32)


def _tc_loss(z1_pad, z2_pad):
    out = pl.pallas_call(
        _tc_loss_body,
        grid=(_NBLK, _NBLK),
        in_specs=[
            pl.BlockSpec((_BR, D), lambda i, j: (i, 0)),
            pl.BlockSpec((_BR, D), lambda i, j: (j, 0)),
        ],
        out_specs=pl.BlockSpec((1, 1), lambda i, j: (0, 0)),
        out_shape=jax.ShapeDtypeStruct((1, 1), jnp.float32),
        scratch_shapes=[
            pltpu.VMEM((_BR, 1), jnp.float32),
            pltpu.VMEM((1, NPAD), jnp.float32),
            pltpu.SMEM((3,), jnp.float32),
        ],
    )(z1_pad, z2_pad)
    return out[0, 0]


def kernel(x_protein, hyperedge_protein_index, x_meta, hyperedge_meta_index,
           pc1_w1, pc1_b1, pc1_g, pc1_be, pc1_rm, pc1_rv, pc1_w2, pc1_b2,
           pc2_w1, pc2_b1, pc2_g, pc2_be, pc2_rm, pc2_rv, pc2_w2, pc2_b2,
           mc1_w1, mc1_b1, mc1_g, mc1_be, mc1_rm, mc1_rv, mc1_w2, mc1_b2,
           mc2_w1, mc2_b1, mc2_g, mc2_be, mc2_rm, mc2_rv, mc2_w2, mc2_b2,
           p12, m13):
    hp = hyperedge_protein_index
    hm = hyperedge_meta_index
    src4 = jnp.stack([hp[0, 0], hp[1, 0], hm[0, 0], hm[1, 0]])
    dst4 = jnp.stack([hp[0, 1], hp[1, 1], hm[0, 1], hm[1, 1]])
    off = (jnp.arange(4, dtype=jnp.int32) * NPAD)[:, None]
    to4d = lambda a: a.reshape(4, NCH, 1, CH)
    src_loc = to4d(src4)
    dst_loc = to4d(dst4)
    src_abs = to4d(src4 + off)
    dst_abs = to4d(dst4 + off)
    ones_ch = jnp.ones((CH,), jnp.float32)

    w1s = jnp.stack([pc1_w1, pc2_w1, mc1_w1, mc2_w1])
    b1s = jnp.stack([pc1_b1, pc2_b1, mc1_b1, mc2_b1])
    gs = jnp.stack([pc1_g, pc2_g, mc1_g, mc2_g])
    bes = jnp.stack([pc1_be, pc2_be, mc1_be, mc2_be])
    rms = jnp.stack([pc1_rm, pc2_rm, mc1_rm, mc2_rm])
    rvs = jnp.stack([pc1_rv, pc2_rv, mc1_rv, mc2_rv])
    w2s = jnp.stack([pc1_w2, pc2_w2, mc1_w2, mc2_w2])
    b2s = jnp.stack([pc1_b2, pc2_b2, mc1_b2, mc2_b2])

    pad = jnp.zeros((NPAD - N, D), jnp.float32)
    x4 = jnp.concatenate([
        x_protein, pad, x_protein, pad, x_meta, pad, x_meta, pad
    ]).reshape(4 * NPAD, D)

    dv, bv = _sc_hist(src_loc, dst_loc, ones_ch)

    a1 = _sc_stage(x4, src_abs, dst_loc)
    b1_ = _tc_scale(a1, bv)
    c1 = _sc_stage(b1_, dst_abs, src_loc)
    z = _tc_mid(c1, dv, w1s, b1s, gs, bes, rms, rvs, w2s)

    a2 = _sc_stage(z, src_abs, dst_loc)
    b2_ = _tc_scale(a2, bv)
    c2 = _sc_stage(b2_, dst_abs, src_loc)
    feats_pad = _tc_final(c2, dv, b2s).reshape(4, NPAD, D)
    feats = feats_pad[:, :N]

    loss_p = _tc_loss(feats_pad[0], feats_pad[1])
    loss_m = _tc_loss(feats_pad[2], feats_pad[3])
    loss = jnp.exp(-p12) * loss_p + p12 + jnp.exp(-m13) * loss_m + m13

    protein = jnp.stack((feats[1], feats[0]))[None]
    meta = jnp.stack((feats[3], feats[2]))[None]
    return (protein, meta, x_protein[None, None], x_meta[None, None], loss)


# speedup vs baseline: 13.1256x; 1.6006x over previous
import functools

import jax
import jax.numpy as jnp
from jax import lax
from jax.experimental import pallas as pl
from jax.experimental.pallas import tpu as pltpu
from jax.experimental.pallas import tpu_sc as plsc

N = 10000
E = 320000
D = 128
CH = 128
NCH = E // CH
NSUB = 16
NCORE = 2
NPAD = 10240
RPT = NPAD // NSUB
ZR = 128
NCHP = 2560
NBLK8 = NCHP // 8
BPT = NBLK8 // NSUB
EP = NCHP * CH


@functools.lru_cache(maxsize=None)
def _sc_mesh():
    return plsc.VectorSubcoreMesh(
        core_axis_name="c", subcore_axis_name="s",
        num_cores=NCORE, num_subcores=NSUB,
    )


def _n_chunks_for(s):
    return (NCH // NSUB) + (s < (NCH % NSUB)).astype(jnp.int32)


def _sc_hist_body(src4, dst4, ones_hbm, dv_out, bv_out, acc_d, acc_b, iv, jv,
                  ones_v, zer_v):
    c = lax.axis_index("c")
    s = lax.axis_index("s")
    nj = _n_chunks_for(s)
    pltpu.sync_copy(ones_hbm, ones_v)

    def _z(i, carry):
        zer_v[pl.ds(i * 16, 16)] = jnp.zeros((16,), jnp.float32)
        return carry
    lax.fori_loop(0, RPT // 16, _z, 0)

    for p in range(2):
        h = 2 * p + c
        pltpu.sync_copy(zer_v, acc_d.at[pl.ds(s * RPT, RPT)])
        pltpu.sync_copy(zer_v, acc_b.at[pl.ds(s * RPT, RPT)])
        plsc.subcore_barrier()

        def _body(j, carry):
            ch = s + NSUB * j
            pltpu.sync_copy(src4.at[h, ch, 0], iv)
            pltpu.sync_copy(dst4.at[h, ch, 0], jv)
            pltpu.sync_copy(ones_v, acc_d.at[iv], add=True)
            pltpu.sync_copy(ones_v, acc_b.at[jv], add=True)
            return carry

        lax.fori_loop(0, nj, _body, 0)
        plsc.subcore_barrier()
        pltpu.sync_copy(acc_d.at[pl.ds(s * RPT, RPT)],
                        dv_out.at[pl.ds(h * NPAD + s * RPT, RPT)])
        pltpu.sync_copy(acc_b.at[pl.ds(s * RPT, RPT)],
                        bv_out.at[pl.ds(h * NPAD + s * RPT, RPT)])
        plsc.subcore_barrier()


@functools.lru_cache(maxsize=None)
def _sc_hist_fn():
    return pl.kernel(
        _sc_hist_body,
        out_type=(
            jax.ShapeDtypeStruct((4 * NPAD,), jnp.float32),
            jax.ShapeDtypeStruct((4 * NPAD,), jnp.float32),
        ),
        mesh=_sc_mesh(),
        scratch_types=[
            pltpu.VMEM_SHARED((NPAD,), jnp.float32),
            pltpu.VMEM_SHARED((NPAD,), jnp.float32),
            pltpu.VMEM((CH,), jnp.int32),
            pltpu.VMEM((CH,), jnp.int32),
            pltpu.VMEM((CH,), jnp.float32),
            pltpu.VMEM((RPT,), jnp.float32),
        ],
    )


def _sc_hist(src4, dst4, ones_ch):
    dv, bv = _sc_hist_fn()(src4, dst4, ones_ch)
    return dv.reshape(4, NPAD), bv.reshape(4, NPAD)


def _sc_stage_body(xin, gidx, sidx, out, acc, gbuf, sbuf, rows0, rows1,
                   sg0, sg1, ss0, ss1):
    c = lax.axis_index("c")
    s = lax.axis_index("s")
    rows = (rows0, rows1)
    sgs = (sg0, sg1)
    sss = (ss0, ss1)

    for p in range(2):
        h = 2 * p + c

        def _z(i, carry):
            r = i // (D // 16)
            k = (i % (D // 16)) * 16
            rows0[r, pl.ds(k, 16)] = jnp.zeros((16,), jnp.float32)
            return carry
        lax.fori_loop(0, ZR * (D // 16), _z, 0)
        for q in range(RPT // ZR):
            pltpu.sync_copy(rows0, acc.at[pl.ds(s * RPT + q * ZR, ZR)])
        plsc.subcore_barrier()

        def _blk(b, carry):
            blk = s * BPT + b
            pltpu.sync_copy(gidx.at[h, blk, 0], gbuf)
            pltpu.sync_copy(sidx.at[h, blk, 0], sbuf)
            d_g = [None, None]
            d_s = [None, None]
            d_g[0] = pltpu.async_copy(xin.at[gbuf.at[0]], rows[0], sgs[0])
            for k in range(8):
                cur = k & 1
                nxt = 1 - cur
                if k + 1 < 8:
                    if k >= 1:
                        d_s[nxt].wait()
                    d_g[nxt] = pltpu.async_copy(
                        xin.at[gbuf.at[k + 1]], rows[nxt], sgs[nxt])
                d_g[cur].wait()
                d_s[cur] = pltpu.async_copy(
                    rows[cur], acc.at[sbuf.at[k]], sss[cur], add=True)
            d_s[0].wait()
            d_s[1].wait()
            return carry

        lax.fori_loop(0, BPT, _blk, 0)
        plsc.subcore_barrier()
        pltpu.sync_copy(
            acc.at[pl.ds(s * RPT, RPT)],
            out.at[pl.ds(h * NPAD + s * RPT, RPT)],
        )
        plsc.subcore_barrier()


@functools.lru_cache(maxsize=None)
def _sc_stage_fn():
    return pl.kernel(
        _sc_stage_body,
        out_type=jax.ShapeDtypeStruct((4 * NPAD, D), jnp.float32),
        mesh=_sc_mesh(),
        scratch_types=[
            pltpu.VMEM_SHARED((NPAD, D), jnp.float32),
            pltpu.VMEM((8, CH), jnp.int32),
            pltpu.VMEM((8, CH), jnp.int32),
            pltpu.VMEM((CH, D), jnp.float32),
            pltpu.VMEM((CH, D), jnp.float32),
            pltpu.SemaphoreType.DMA,
            pltpu.SemaphoreType.DMA,
            pltpu.SemaphoreType.DMA,
            pltpu.SemaphoreType.DMA,
        ],
    )


def _sc_stage(xin_flat, gidx_abs, sidx_loc):
    return _sc_stage_fn()(xin_flat, gidx_abs, sidx_loc)


def _tc_scale_body(a_ref, bv_ref, o_ref):
    bv = bv_ref[0, 0, :]
    inv = jnp.where(bv > 0, 1.0 / bv, 0.0)
    o_ref[...] = a_ref[...] * inv[:, None]


def _tc_scale(a_flat, bv):
    return pl.pallas_call(
        _tc_scale_body,
        grid=(4,),
        in_specs=[
            pl.BlockSpec((NPAD, D), lambda i: (i, 0)),
            pl.BlockSpec((1, 1, NPAD), lambda i: (i, 0, 0)),
        ],
        out_specs=pl.BlockSpec((NPAD, D), lambda i: (i, 0)),
        out_shape=jax.ShapeDtypeStruct((4 * NPAD, D), jnp.float32),
    )(a_flat, bv.reshape(4, 1, NPAD))


def _tc_mid_body(c_ref, dv_ref, w1_ref, b1_ref, g_ref, be_ref, rm_ref, rv_ref,
                 w2_ref, o_ref):
    dv = dv_ref[0, 0, :]
    dinv = jnp.where(dv > 0, 1.0 / dv, 0.0)
    sg = g_ref[0, 0, :] / jnp.sqrt(rv_ref[0, 0, :] + 1e-5)
    t = be_ref[0, 0, :] - rm_ref[0, 0, :] * sg
    w1 = w1_ref[0]
    w2 = w2_ref[0]
    w1s = w1 * sg[:, None]
    weff = lax.dot_general(w1s, w2, (((0,), (1,)), ((), ())),
                           preferred_element_type=jnp.float32)
    bvec = (b1_ref[0, 0, :] * sg + t)[None, :]
    beff = lax.dot_general(bvec, w2, (((1,), (1,)), ((), ())),
                           preferred_element_type=jnp.float32)
    cd = c_ref[...] * dinv[:, None]
    o_ref[...] = lax.dot_general(cd, weff, (((1,), (0,)), ((), ())),
                                 preferred_element_type=jnp.float32) + beff


def _tc_mid(c_flat, dv, w1s_, b1s_, gs_, bes_, rms_, rvs_, w2s_):
    vec = lambda x: x.reshape(4, 1, 2 * D)
    return pl.pallas_call(
        _tc_mid_body,
        grid=(4,),
        in_specs=[
            pl.BlockSpec((NPAD, D), lambda i: (i, 0)),
            pl.BlockSpec((1, 1, NPAD), lambda i: (i, 0, 0)),
            pl.BlockSpec((1, 2 * D, D), lambda i: (i, 0, 0)),
            pl.BlockSpec((1, 1, 2 * D), lambda i: (i, 0, 0)),
            pl.BlockSpec((1, 1, 2 * D), lambda i: (i, 0, 0)),
            pl.BlockSpec((1, 1, 2 * D), lambda i: (i, 0, 0)),
            pl.BlockSpec((1, 1, 2 * D), lambda i: (i, 0, 0)),
            pl.BlockSpec((1, 1, 2 * D), lambda i: (i, 0, 0)),
            pl.BlockSpec((1, D, 2 * D), lambda i: (i, 0, 0)),
        ],
        out_specs=pl.BlockSpec((NPAD, D), lambda i: (i, 0)),
        out_shape=jax.ShapeDtypeStruct((4 * NPAD, D), jnp.float32),
    )(c_flat, dv.reshape(4, 1, NPAD), w1s_, vec(b1s_), vec(gs_), vec(bes_),
      vec(rms_), vec(rvs_), w2s_)


def _tc_final_body(c_ref, dv_ref, b2_ref, o_ref):
    dv = dv_ref[0, 0, :]
    dinv = jnp.where(dv > 0, 1.0 / dv, 0.0)
    o_ref[...] = c_ref[...] * dinv[:, None] + b2_ref[0, 0, :][None, :]


def _tc_final(c_flat, dv, b2s_):
    return pl.pallas_call(
        _tc_final_body,
        grid=(4,),
        in_specs=[
            pl.BlockSpec((NPAD, D), lambda i: (i, 0)),
            pl.BlockSpec((1, 1, NPAD), lambda i: (i, 0, 0)),
            pl.BlockSpec((1, 1, D), lambda i: (i, 0, 0)),
        ],
        out_specs=pl.BlockSpec((NPAD, D), lambda i: (i, 0)),
        out_shape=jax.ShapeDtypeStruct((4 * NPAD, D), jnp.float32),
    )(c_flat, dv.reshape(4, 1, NPAD), b2s_.reshape(4, 1, D))


_BR = 1024
_NBLK = NPAD // _BR


def _tc_loss_body(z1_ref, z2_ref, o_ref, sr, sc_, acc):
    i = pl.program_id(0)
    j = pl.program_id(1)

    @pl.when(jnp.logical_and(i == 0, j == 0))
    def _():
        acc[0] = 0.0
        acc[1] = 0.0
        acc[2] = 0.0

    z1 = z1_ref[...]
    z2 = z2_ref[...]
    n1 = z1 / jnp.maximum(
        jnp.sqrt(jnp.sum(z1 * z1, axis=1, keepdims=True)), 1e-12)
    n2 = z2 / jnp.maximum(
        jnp.sqrt(jnp.sum(z2 * z2, axis=1, keepdims=True)), 1e-12)
    s_blk = lax.dot_general(n1, n2, (((1,), (1,)), ((), ())),
                            preferred_element_type=jnp.float32) * (1.0 / 0.7)
    ri = lax.broadcasted_iota(jnp.int32, (_BR, _BR), 0)
    ci = lax.broadcasted_iota(jnp.int32, (_BR, _BR), 1)
    valid = jnp.logical_and(i * _BR + ri < N, j * _BR + ci < N)
    p_blk = jnp.where(valid, jnp.exp(s_blk), 0.0)

    @pl.when(j == 0)
    def _():
        sr[...] = jnp.zeros_like(sr)

    sr[...] = sr[...] + jnp.sum(p_blk, axis=1, keepdims=True)

    @pl.when(i == 0)
    def _():
        sc_[0, pl.ds(j * _BR, _BR)] = jnp.zeros((_BR,), jnp.float32)

    cur = sc_[0, pl.ds(j * _BR, _BR)]
    sc_[0, pl.ds(j * _BR, _BR)] = cur + jnp.sum(p_blk, axis=0)

    @pl.when(i == j)
    def _():
        dmask = jnp.logical_and(ri == ci, valid)
        acc[2] = acc[2] + jnp.sum(jnp.where(dmask, s_blk, 0.0))

    @pl.when(j == _NBLK - 1)
    def _():
        rv_ = lax.broadcasted_iota(jnp.int32, (_BR, 1), 0) + i * _BR < N
        acc[0] = acc[0] + jnp.sum(jnp.where(rv_, jnp.log(sr[...]), 0.0))

    @pl.when(jnp.logical_and(i == _NBLK - 1, j == _NBLK - 1))
    def _():
        cv_ = lax.broadcasted_iota(jnp.int32, (1, NPAD), 1) < N
        acc[1] = jnp.sum(jnp.where(cv_, jnp.log(sc_[...]), 0.0))
        val = 0.5 * ((acc[0] - acc[2]) + (acc[1] - acc[2])) / N
        o_ref[...] = jnp.full((1, 1), val, jnp.float32)


def _tc_loss(z1_pad, z2_pad):
    out = pl.pallas_call(
        _tc_loss_body,
        grid=(_NBLK, _NBLK),
        in_specs=[
            pl.BlockSpec((_BR, D), lambda i, j: (i, 0)),
            pl.BlockSpec((_BR, D), lambda i, j: (j, 0)),
        ],
        out_specs=pl.BlockSpec((1, 1), lambda i, j: (0, 0)),
        out_shape=jax.ShapeDtypeStruct((1, 1), jnp.float32),
        scratch_shapes=[
            pltpu.VMEM((_BR, 1), jnp.float32),
            pltpu.VMEM((1, NPAD), jnp.float32),
            pltpu.SMEM((3,), jnp.float32),
        ],
    )(z1_pad, z2_pad)
    return out[0, 0]


def kernel(x_protein, hyperedge_protein_index, x_meta, hyperedge_meta_index,
           pc1_w1, pc1_b1, pc1_g, pc1_be, pc1_rm, pc1_rv, pc1_w2, pc1_b2,
           pc2_w1, pc2_b1, pc2_g, pc2_be, pc2_rm, pc2_rv, pc2_w2, pc2_b2,
           mc1_w1, mc1_b1, mc1_g, mc1_be, mc1_rm, mc1_rv, mc1_w2, mc1_b2,
           mc2_w1, mc2_b1, mc2_g, mc2_be, mc2_rm, mc2_rv, mc2_w2, mc2_b2,
           p12, m13):
    hp = hyperedge_protein_index
    hm = hyperedge_meta_index
    src4 = jnp.stack([hp[0, 0], hp[1, 0], hm[0, 0], hm[1, 0]])
    dst4 = jnp.stack([hp[0, 1], hp[1, 1], hm[0, 1], hm[1, 1]])
    off = (jnp.arange(4, dtype=jnp.int32) * NPAD)[:, None]
    to4d = lambda a: a.reshape(4, NCH, 1, CH)
    src_loc = to4d(src4)
    dst_loc = to4d(dst4)
    e_pad = jnp.arange(EP - E, dtype=jnp.int32)
    gpad = (e_pad % N)[None, :] + off
    spad = jnp.broadcast_to(N + (e_pad % (NPAD - N)), (4, EP - E)).astype(jnp.int32)
    to5d = lambda a: a.reshape(4, NBLK8, 1, 8, CH)
    src_abs5 = to5d(jnp.concatenate([src4 + off, gpad], axis=1))
    dst_abs5 = to5d(jnp.concatenate([dst4 + off, gpad], axis=1))
    src_loc5 = to5d(jnp.concatenate([src4, spad], axis=1))
    dst_loc5 = to5d(jnp.concatenate([dst4, spad], axis=1))
    ones_ch = jnp.ones((CH,), jnp.float32)

    w1s = jnp.stack([pc1_w1, pc2_w1, mc1_w1, mc2_w1])
    b1s = jnp.stack([pc1_b1, pc2_b1, mc1_b1, mc2_b1])
    gs = jnp.stack([pc1_g, pc2_g, mc1_g, mc2_g])
    bes = jnp.stack([pc1_be, pc2_be, mc1_be, mc2_be])
    rms = jnp.stack([pc1_rm, pc2_rm, mc1_rm, mc2_rm])
    rvs = jnp.stack([pc1_rv, pc2_rv, mc1_rv, mc2_rv])
    w2s = jnp.stack([pc1_w2, pc2_w2, mc1_w2, mc2_w2])
    b2s = jnp.stack([pc1_b2, pc2_b2, mc1_b2, mc2_b2])

    pad = jnp.zeros((NPAD - N, D), jnp.float32)
    x4 = jnp.concatenate([
        x_protein, pad, x_protein, pad, x_meta, pad, x_meta, pad
    ]).reshape(4 * NPAD, D)

    dv, bv = _sc_hist(src_loc, dst_loc, ones_ch)

    a1 = _sc_stage(x4, src_abs5, dst_loc5)
    b1_ = _tc_scale(a1, bv)
    c1 = _sc_stage(b1_, dst_abs5, src_loc5)
    z = _tc_mid(c1, dv, w1s, b1s, gs, bes, rms, rvs, w2s)

    a2 = _sc_stage(z, src_abs5, dst_loc5)
    b2_ = _tc_scale(a2, bv)
    c2 = _sc_stage(b2_, dst_abs5, src_loc5)
    feats_pad = _tc_final(c2, dv, b2s).reshape(4, NPAD, D)
    feats = feats_pad[:, :N]

    loss_p = _tc_loss(feats_pad[0], feats_pad[1])
    loss_m = _tc_loss(feats_pad[2], feats_pad[3])
    loss = jnp.exp(-p12) * loss_p + p12 + jnp.exp(-m13) * loss_m + m13

    protein = jnp.stack((feats[1], feats[0]))[None]
    meta = jnp.stack((feats[3], feats[2]))[None]
    return (protein, meta, x_protein[None, None], x_meta[None, None], loss)


# speedup vs baseline: 16.4561x; 1.2537x over previous
import functools

import jax
import jax.numpy as jnp
from jax import lax
from jax.experimental import pallas as pl
from jax.experimental.pallas import tpu as pltpu
from jax.experimental.pallas import tpu_sc as plsc

N = 10000
E = 320000
D = 128
CH = 128
NCH = E // CH
NSUB = 16
NCORE = 2
NPAD = 10240
RPT = NPAD // NSUB
ZR = 128
EP = 327680
CHS = 64
NBLK8 = EP // (8 * CHS)
BPT = NBLK8 // NSUB


@functools.lru_cache(maxsize=None)
def _sc_mesh():
    return plsc.VectorSubcoreMesh(
        core_axis_name="c", subcore_axis_name="s",
        num_cores=NCORE, num_subcores=NSUB,
    )


def _n_chunks_for(s):
    return (NCH // NSUB) + (s < (NCH % NSUB)).astype(jnp.int32)


def _sc_hist_body(src4, dst4, ones_hbm, dv_out, bv_out, acc_d, acc_b, iv, jv,
                  ones_v, zer_v):
    c = lax.axis_index("c")
    s = lax.axis_index("s")
    nj = _n_chunks_for(s)
    pltpu.sync_copy(ones_hbm, ones_v)

    def _z(i, carry):
        zer_v[pl.ds(i * 16, 16)] = jnp.zeros((16,), jnp.float32)
        return carry
    lax.fori_loop(0, RPT // 16, _z, 0)

    for p in range(2):
        h = 2 * p + c
        pltpu.sync_copy(zer_v, acc_d.at[pl.ds(s * RPT, RPT)])
        pltpu.sync_copy(zer_v, acc_b.at[pl.ds(s * RPT, RPT)])
        plsc.subcore_barrier()

        def _body(j, carry):
            ch = s + NSUB * j
            pltpu.sync_copy(src4.at[h, ch, 0], iv)
            pltpu.sync_copy(dst4.at[h, ch, 0], jv)
            pltpu.sync_copy(ones_v, acc_d.at[iv], add=True)
            pltpu.sync_copy(ones_v, acc_b.at[jv], add=True)
            return carry

        lax.fori_loop(0, nj, _body, 0)
        plsc.subcore_barrier()
        pltpu.sync_copy(acc_d.at[pl.ds(s * RPT, RPT)],
                        dv_out.at[pl.ds(h * NPAD + s * RPT, RPT)])
        pltpu.sync_copy(acc_b.at[pl.ds(s * RPT, RPT)],
                        bv_out.at[pl.ds(h * NPAD + s * RPT, RPT)])
        plsc.subcore_barrier()


@functools.lru_cache(maxsize=None)
def _sc_hist_fn():
    return pl.kernel(
        _sc_hist_body,
        out_type=(
            jax.ShapeDtypeStruct((4 * NPAD,), jnp.float32),
            jax.ShapeDtypeStruct((4 * NPAD,), jnp.float32),
        ),
        mesh=_sc_mesh(),
        scratch_types=[
            pltpu.VMEM_SHARED((NPAD,), jnp.float32),
            pltpu.VMEM_SHARED((NPAD,), jnp.float32),
            pltpu.VMEM((CH,), jnp.int32),
            pltpu.VMEM((CH,), jnp.int32),
            pltpu.VMEM((CH,), jnp.float32),
            pltpu.VMEM((RPT,), jnp.float32),
        ],
    )


def _sc_hist(src4, dst4, ones_ch):
    dv, bv = _sc_hist_fn()(src4, dst4, ones_ch)
    return dv.reshape(4, NPAD), bv.reshape(4, NPAD)


_NB_ROWS = 5
_QG = 3


def _sc_stage_body(xin, gidx, sidx, out, acc, gb2, sb2, rows6,
                   sgat, ssca, sidxs):
    c = lax.axis_index("c")
    s = lax.axis_index("s")
    nch_t = 8 * BPT

    for p in range(2):
        h = 2 * p + c

        def _z(i, carry):
            r = i // (D // 16)
            k = (i % (D // 16)) * 16
            rows6[0, r, pl.ds(k, 16)] = jnp.zeros((16,), jnp.float32)
            return carry
        lax.fori_loop(0, CHS * (D // 16), _z, 0)
        for q in range(RPT // CHS):
            pltpu.sync_copy(rows6.at[0], acc.at[pl.ds(s * RPT + q * CHS, CHS)])
        plsc.subcore_barrier()

        base_blk = s * BPT
        pltpu.sync_copy(gidx.at[h, base_blk, 0], gb2.at[0])
        pltpu.sync_copy(sidx.at[h, base_blk, 0], sb2.at[0])
        for kk in range(_QG):
            pltpu.async_copy(xin.at[gb2.at[0, kk]], rows6.at[kk], sgat)

        def _chunk(k, carry):
            blk = k // 8
            sel = lax.rem(blk, 2)
            koff = lax.rem(k, 8)
            kq = k + _QG

            @pl.when(koff == 5)
            def _():
                nsel = lax.rem(blk + 1, 2)
                pltpu.make_async_copy(
                    gidx.at[h, base_blk, 0], gb2.at[nsel], sidxs).wait()
                pltpu.make_async_copy(
                    sidx.at[h, base_blk, 0], sb2.at[nsel], sidxs).wait()

            @pl.when(kq < nch_t)
            def _():
                @pl.when(kq >= _NB_ROWS)
                def _():
                    pltpu.make_async_copy(
                        rows6.at[0], acc.at[sb2.at[0, 0]], ssca).wait()
                qsel = lax.rem(kq // 8, 2)
                qoff = lax.rem(kq, 8)
                pltpu.async_copy(
                    xin.at[gb2.at[qsel, qoff]],
                    rows6.at[lax.rem(kq, _NB_ROWS)], sgat)

            @pl.when(koff == 2)
            def _():
                nb = jnp.minimum(blk + 1, BPT - 1)
                nsel = lax.rem(blk + 1, 2)
                pltpu.async_copy(gidx.at[h, base_blk + nb, 0],
                                 gb2.at[nsel], sidxs)
                pltpu.async_copy(sidx.at[h, base_blk + nb, 0],
                                 sb2.at[nsel], sidxs)

            pltpu.make_async_copy(
                xin.at[gb2.at[sel, koff]],
                rows6.at[lax.rem(k, _NB_ROWS)], sgat).wait()
            pltpu.async_copy(
                rows6.at[lax.rem(k, _NB_ROWS)],
                acc.at[sb2.at[sel, koff]], ssca, add=True)
            return carry

        lax.fori_loop(0, nch_t, _chunk, 0)
        for _ in range(_NB_ROWS):
            pltpu.make_async_copy(
                rows6.at[0], acc.at[sb2.at[0, 0]], ssca).wait()
        plsc.subcore_barrier()
        pltpu.sync_copy(
            acc.at[pl.ds(s * RPT, RPT)],
            out.at[pl.ds(h * NPAD + s * RPT, RPT)],
        )
        plsc.subcore_barrier()


@functools.lru_cache(maxsize=None)
def _sc_stage_fn():
    return pl.kernel(
        _sc_stage_body,
        out_type=jax.ShapeDtypeStruct((4 * NPAD, D), jnp.float32),
        mesh=_sc_mesh(),
        scratch_types=[
            pltpu.VMEM_SHARED((NPAD, D), jnp.float32),
            pltpu.VMEM((2, 8, CHS), jnp.int32),
            pltpu.VMEM((2, 8, CHS), jnp.int32),
            pltpu.VMEM((_NB_ROWS, CHS, D), jnp.float32),
            pltpu.SemaphoreType.DMA,
            pltpu.SemaphoreType.DMA,
            pltpu.SemaphoreType.DMA,
        ],
    )


def _sc_stage(xin_flat, gidx_abs, sidx_loc):
    return _sc_stage_fn()(xin_flat, gidx_abs, sidx_loc)


def _tc_scale_body(a_ref, bv_ref, o_ref):
    bv = bv_ref[0, 0, :]
    inv = jnp.where(bv > 0, 1.0 / bv, 0.0)
    o_ref[...] = a_ref[...] * inv[:, None]


def _tc_scale(a_flat, bv):
    return pl.pallas_call(
        _tc_scale_body,
        grid=(4,),
        in_specs=[
            pl.BlockSpec((NPAD, D), lambda i: (i, 0)),
            pl.BlockSpec((1, 1, NPAD), lambda i: (i, 0, 0)),
        ],
        out_specs=pl.BlockSpec((NPAD, D), lambda i: (i, 0)),
        out_shape=jax.ShapeDtypeStruct((4 * NPAD, D), jnp.float32),
    )(a_flat, bv.reshape(4, 1, NPAD))


def _tc_mid_body(c_ref, dv_ref, w1_ref, b1_ref, g_ref, be_ref, rm_ref, rv_ref,
                 w2_ref, o_ref):
    dv = dv_ref[0, 0, :]
    dinv = jnp.where(dv > 0, 1.0 / dv, 0.0)
    sg = g_ref[0, 0, :] / jnp.sqrt(rv_ref[0, 0, :] + 1e-5)
    t = be_ref[0, 0, :] - rm_ref[0, 0, :] * sg
    w1 = w1_ref[0]
    w2 = w2_ref[0]
    w1s = w1 * sg[:, None]
    weff = lax.dot_general(w1s, w2, (((0,), (1,)), ((), ())),
                           preferred_element_type=jnp.float32)
    bvec = (b1_ref[0, 0, :] * sg + t)[None, :]
    beff = lax.dot_general(bvec, w2, (((1,), (1,)), ((), ())),
                           preferred_element_type=jnp.float32)
    cd = c_ref[...] * dinv[:, None]
    o_ref[...] = lax.dot_general(cd, weff, (((1,), (0,)), ((), ())),
                                 preferred_element_type=jnp.float32) + beff


def _tc_mid(c_flat, dv, w1s_, b1s_, gs_, bes_, rms_, rvs_, w2s_):
    vec = lambda x: x.reshape(4, 1, 2 * D)
    return pl.pallas_call(
        _tc_mid_body,
        grid=(4,),
        in_specs=[
            pl.BlockSpec((NPAD, D), lambda i: (i, 0)),
            pl.BlockSpec((1, 1, NPAD), lambda i: (i, 0, 0)),
            pl.BlockSpec((1, 2 * D, D), lambda i: (i, 0, 0)),
            pl.BlockSpec((1, 1, 2 * D), lambda i: (i, 0, 0)),
            pl.BlockSpec((1, 1, 2 * D), lambda i: (i, 0, 0)),
            pl.BlockSpec((1, 1, 2 * D), lambda i: (i, 0, 0)),
            pl.BlockSpec((1, 1, 2 * D), lambda i: (i, 0, 0)),
            pl.BlockSpec((1, 1, 2 * D), lambda i: (i, 0, 0)),
            pl.BlockSpec((1, D, 2 * D), lambda i: (i, 0, 0)),
        ],
        out_specs=pl.BlockSpec((NPAD, D), lambda i: (i, 0)),
        out_shape=jax.ShapeDtypeStruct((4 * NPAD, D), jnp.float32),
    )(c_flat, dv.reshape(4, 1, NPAD), w1s_, vec(b1s_), vec(gs_), vec(bes_),
      vec(rms_), vec(rvs_), w2s_)


def _tc_final_body(c_ref, dv_ref, b2_ref, o_ref):
    dv = dv_ref[0, 0, :]
    dinv = jnp.where(dv > 0, 1.0 / dv, 0.0)
    o_ref[...] = c_ref[...] * dinv[:, None] + b2_ref[0, 0, :][None, :]


def _tc_final(c_flat, dv, b2s_):
    return pl.pallas_call(
        _tc_final_body,
        grid=(4,),
        in_specs=[
            pl.BlockSpec((NPAD, D), lambda i: (i, 0)),
            pl.BlockSpec((1, 1, NPAD), lambda i: (i, 0, 0)),
            pl.BlockSpec((1, 1, D), lambda i: (i, 0, 0)),
        ],
        out_specs=pl.BlockSpec((NPAD, D), lambda i: (i, 0)),
        out_shape=jax.ShapeDtypeStruct((4 * NPAD, D), jnp.float32),
    )(c_flat, dv.reshape(4, 1, NPAD), b2s_.reshape(4, 1, D))


_BR = 1024
_NBLK = NPAD // _BR


def _tc_loss_body(z1_ref, z2_ref, o_ref, sr, sc_, acc):
    i = pl.program_id(0)
    j = pl.program_id(1)

    @pl.when(jnp.logical_and(i == 0, j == 0))
    def _():
        acc[0] = 0.0
        acc[1] = 0.0
        acc[2] = 0.0

    z1 = z1_ref[...]
    z2 = z2_ref[...]
    n1 = z1 / jnp.maximum(
        jnp.sqrt(jnp.sum(z1 * z1, axis=1, keepdims=True)), 1e-12)
    n2 = z2 / jnp.maximum(
        jnp.sqrt(jnp.sum(z2 * z2, axis=1, keepdims=True)), 1e-12)
    s_blk = lax.dot_general(n1, n2, (((1,), (1,)), ((), ())),
                            preferred_element_type=jnp.float32) * (1.0 / 0.7)
    ri = lax.broadcasted_iota(jnp.int32, (_BR, _BR), 0)
    ci = lax.broadcasted_iota(jnp.int32, (_BR, _BR), 1)
    valid = jnp.logical_and(i * _BR + ri < N, j * _BR + ci < N)
    p_blk = jnp.where(valid, jnp.exp(s_blk), 0.0)

    @pl.when(j == 0)
    def _():
        sr[...] = jnp.zeros_like(sr)

    sr[...] = sr[...] + jnp.sum(p_blk, axis=1, keepdims=True)

    @pl.when(i == 0)
    def _():
        sc_[0, pl.ds(j * _BR, _BR)] = jnp.zeros((_BR,), jnp.float32)

    cur = sc_[0, pl.ds(j * _BR, _BR)]
    sc_[0, pl.ds(j * _BR, _BR)] = cur + jnp.sum(p_blk, axis=0)

    @pl.when(i == j)
    def _():
        dmask = jnp.logical_and(ri == ci, valid)
        acc[2] = acc[2] + jnp.sum(jnp.where(dmask, s_blk, 0.0))

    @pl.when(j == _NBLK - 1)
    def _():
        rv_ = lax.broadcasted_iota(jnp.int32, (_BR, 1), 0) + i * _BR < N
        acc[0] = acc[0] + jnp.sum(jnp.where(rv_, jnp.log(sr[...]), 0.0))

    @pl.when(jnp.logical_and(i == _NBLK - 1, j == _NBLK - 1))
    def _():
        cv_ = lax.broadcasted_iota(jnp.int32, (1, NPAD), 1) < N
        acc[1] = jnp.sum(jnp.where(cv_, jnp.log(sc_[...]), 0.0))
        val = 0.5 * ((acc[0] - acc[2]) + (acc[1] - acc[2])) / N
        o_ref[...] = jnp.full((1, 1), val, jnp.float32)


def _tc_loss(z1_pad, z2_pad):
    out = pl.pallas_call(
        _tc_loss_body,
        grid=(_NBLK, _NBLK),
        in_specs=[
            pl.BlockSpec((_BR, D), lambda i, j: (i, 0)),
            pl.BlockSpec((_BR, D), lambda i, j: (j, 0)),
        ],
        out_specs=pl.BlockSpec((1, 1), lambda i, j: (0, 0)),
        out_shape=jax.ShapeDtypeStruct((1, 1), jnp.float32),
        scratch_shapes=[
            pltpu.VMEM((_BR, 1), jnp.float32),
            pltpu.VMEM((1, NPAD), jnp.float32),
            pltpu.SMEM((3,), jnp.float32),
        ],
    )(z1_pad, z2_pad)
    return out[0, 0]


def kernel(x_protein, hyperedge_protein_index, x_meta, hyperedge_meta_index,
           pc1_w1, pc1_b1, pc1_g, pc1_be, pc1_rm, pc1_rv, pc1_w2, pc1_b2,
           pc2_w1, pc2_b1, pc2_g, pc2_be, pc2_rm, pc2_rv, pc2_w2, pc2_b2,
           mc1_w1, mc1_b1, mc1_g, mc1_be, mc1_rm, mc1_rv, mc1_w2, mc1_b2,
           mc2_w1, mc2_b1, mc2_g, mc2_be, mc2_rm, mc2_rv, mc2_w2, mc2_b2,
           p12, m13):
    hp = hyperedge_protein_index
    hm = hyperedge_meta_index
    src4 = jnp.stack([hp[0, 0], hp[1, 0], hm[0, 0], hm[1, 0]])
    dst4 = jnp.stack([hp[0, 1], hp[1, 1], hm[0, 1], hm[1, 1]])
    off = (jnp.arange(4, dtype=jnp.int32) * NPAD)[:, None]
    to4d = lambda a: a.reshape(4, NCH, 1, CH)
    src_loc = to4d(src4)
    dst_loc = to4d(dst4)
    e_pad = jnp.arange(EP - E, dtype=jnp.int32)
    gpad = (e_pad % N)[None, :] + off
    spad = jnp.broadcast_to(N + (e_pad % (NPAD - N)), (4, EP - E)).astype(jnp.int32)
    to5d = lambda a: a.reshape(4, NBLK8, 1, 8, CHS)
    src_abs5 = to5d(jnp.concatenate([src4 + off, gpad], axis=1))
    dst_abs5 = to5d(jnp.concatenate([dst4 + off, gpad], axis=1))
    src_loc5 = to5d(jnp.concatenate([src4, spad], axis=1))
    dst_loc5 = to5d(jnp.concatenate([dst4, spad], axis=1))
    ones_ch = jnp.ones((CH,), jnp.float32)

    w1s = jnp.stack([pc1_w1, pc2_w1, mc1_w1, mc2_w1])
    b1s = jnp.stack([pc1_b1, pc2_b1, mc1_b1, mc2_b1])
    gs = jnp.stack([pc1_g, pc2_g, mc1_g, mc2_g])
    bes = jnp.stack([pc1_be, pc2_be, mc1_be, mc2_be])
    rms = jnp.stack([pc1_rm, pc2_rm, mc1_rm, mc2_rm])
    rvs = jnp.stack([pc1_rv, pc2_rv, mc1_rv, mc2_rv])
    w2s = jnp.stack([pc1_w2, pc2_w2, mc1_w2, mc2_w2])
    b2s = jnp.stack([pc1_b2, pc2_b2, mc1_b2, mc2_b2])

    pad = jnp.zeros((NPAD - N, D), jnp.float32)
    x4 = jnp.concatenate([
        x_protein, pad, x_protein, pad, x_meta, pad, x_meta, pad
    ]).reshape(4 * NPAD, D)

    dv, bv = _sc_hist(src_loc, dst_loc, ones_ch)

    a1 = _sc_stage(x4, src_abs5, dst_loc5)
    b1_ = _tc_scale(a1, bv)
    c1 = _sc_stage(b1_, dst_abs5, src_loc5)
    z = _tc_mid(c1, dv, w1s, b1s, gs, bes, rms, rvs, w2s)

    a2 = _sc_stage(z, src_abs5, dst_loc5)
    b2_ = _tc_scale(a2, bv)
    c2 = _sc_stage(b2_, dst_abs5, src_loc5)
    feats_pad = _tc_final(c2, dv, b2s).reshape(4, NPAD, D)
    feats = feats_pad[:, :N]

    loss_p = _tc_loss(feats_pad[0], feats_pad[1])
    loss_m = _tc_loss(feats_pad[2], feats_pad[3])
    loss = jnp.exp(-p12) * loss_p + p12 + jnp.exp(-m13) * loss_m + m13

    protein = jnp.stack((feats[1], feats[0]))[None]
    meta = jnp.stack((feats[3], feats[2]))[None]
    return (protein, meta, x_protein[None, None], x_meta[None, None], loss)


# speedup vs baseline: 18.6030x; 1.1305x over previous
import functools

import jax
import jax.numpy as jnp
from jax import lax
from jax.experimental import pallas as pl
from jax.experimental.pallas import tpu as pltpu
from jax.experimental.pallas import tpu_sc as plsc

N = 10000
E = 320000
D = 128
NSUB = 16
NCORE = 2
NPAD = 10240
RPT = NPAD // NSUB
ZR = 128
EP = 327680
CHS = 64
NBLK8 = EP // (8 * CHS)
BPT = NBLK8 // NSUB


@functools.lru_cache(maxsize=None)
def _sc_mesh():
    return plsc.VectorSubcoreMesh(
        core_axis_name="c", subcore_axis_name="s",
        num_cores=NCORE, num_subcores=NSUB,
    )


def _sc_hist_body(src5, dst5, ones_hbm, dv_out, bv_out, acc_d, acc_b,
                  gb2, sb2, ones_v, zbuf, ssca, sidxs):
    c = lax.axis_index("c")
    s = lax.axis_index("s")
    nch_t = 8 * BPT
    pltpu.sync_copy(ones_hbm, ones_v)

    def _z(i, carry):
        zbuf[pl.ds(i * 16, 16)] = jnp.zeros((16,), jnp.float32)
        return carry
    lax.fori_loop(0, RPT // 16, _z, 0)

    for p in range(2):
        h = 2 * p + c
        pltpu.sync_copy(zbuf, acc_d.at[pl.ds(s * RPT, RPT)])
        pltpu.sync_copy(zbuf, acc_b.at[pl.ds(s * RPT, RPT)])
        plsc.subcore_barrier()

        base_blk = s * BPT
        pltpu.sync_copy(src5.at[h, base_blk, 0], gb2.at[0])
        pltpu.sync_copy(dst5.at[h, base_blk, 0], sb2.at[0])

        def _chunk(k, carry):
            blk = k // 8
            sel = lax.rem(blk, 2)
            koff = lax.rem(k, 8)

            @pl.when(koff == 5)
            def _():
                nsel = lax.rem(blk + 1, 2)
                pltpu.make_async_copy(
                    src5.at[h, base_blk, 0], gb2.at[nsel], sidxs).wait()
                pltpu.make_async_copy(
                    dst5.at[h, base_blk, 0], sb2.at[nsel], sidxs).wait()

            @pl.when(k >= 3)
            def _():
                pltpu.make_async_copy(
                    ones_v, acc_d.at[gb2.at[0, 0]], ssca).wait()
                pltpu.make_async_copy(
                    ones_v, acc_b.at[sb2.at[0, 0]], ssca).wait()

            @pl.when(koff == 2)
            def _():
                nb = jnp.minimum(blk + 1, BPT - 1)
                nsel = lax.rem(blk + 1, 2)
                pltpu.async_copy(src5.at[h, base_blk + nb, 0],
                                 gb2.at[nsel], sidxs)
                pltpu.async_copy(dst5.at[h, base_blk + nb, 0],
                                 sb2.at[nsel], sidxs)

            pltpu.async_copy(ones_v, acc_d.at[gb2.at[sel, koff]], ssca,
                             add=True)
            pltpu.async_copy(ones_v, acc_b.at[sb2.at[sel, koff]], ssca,
                             add=True)
            return carry

        lax.fori_loop(0, nch_t, _chunk, 0)
        for _ in range(3):
            pltpu.make_async_copy(ones_v, acc_d.at[gb2.at[0, 0]], ssca).wait()
            pltpu.make_async_copy(ones_v, acc_b.at[sb2.at[0, 0]], ssca).wait()
        plsc.subcore_barrier()
        pltpu.sync_copy(acc_d.at[pl.ds(s * RPT, RPT)],
                        dv_out.at[pl.ds(h * NPAD + s * RPT, RPT)])
        pltpu.sync_copy(acc_b.at[pl.ds(s * RPT, RPT)],
                        bv_out.at[pl.ds(h * NPAD + s * RPT, RPT)])
        plsc.subcore_barrier()


@functools.lru_cache(maxsize=None)
def _sc_hist_fn():
    return pl.kernel(
        _sc_hist_body,
        out_type=(
            jax.ShapeDtypeStruct((4 * NPAD,), jnp.float32),
            jax.ShapeDtypeStruct((4 * NPAD,), jnp.float32),
        ),
        mesh=_sc_mesh(),
        scratch_types=[
            pltpu.VMEM_SHARED((NPAD,), jnp.float32),
            pltpu.VMEM_SHARED((NPAD,), jnp.float32),
            pltpu.VMEM((2, 8, CHS), jnp.int32),
            pltpu.VMEM((2, 8, CHS), jnp.int32),
            pltpu.VMEM((CHS,), jnp.float32),
            pltpu.VMEM((RPT,), jnp.float32),
            pltpu.SemaphoreType.DMA,
            pltpu.SemaphoreType.DMA,
        ],
    )


def _sc_hist(src5, dst5, ones_ch):
    dv, bv = _sc_hist_fn()(src5, dst5, ones_ch)
    return dv.reshape(4, NPAD), bv.reshape(4, NPAD)


_NB_ROWS = 5
_QG = 3


def _sc_stage_body(xin, gidx, sidx, out, acc, gb2, sb2, rows6,
                   sgat, ssca, sidxs):
    c = lax.axis_index("c")
    s = lax.axis_index("s")
    nch_t = 8 * BPT

    for p in range(2):
        h = 2 * p + c

        def _z(i, carry):
            r = i // (D // 16)
            k = (i % (D // 16)) * 16
            rows6[0, r, pl.ds(k, 16)] = jnp.zeros((16,), jnp.float32)
            return carry
        lax.fori_loop(0, CHS * (D // 16), _z, 0)
        for q in range(RPT // CHS):
            pltpu.sync_copy(rows6.at[0], acc.at[pl.ds(s * RPT + q * CHS, CHS)])
        plsc.subcore_barrier()

        base_blk = s * BPT
        pltpu.sync_copy(gidx.at[h, base_blk, 0], gb2.at[0])
        pltpu.sync_copy(sidx.at[h, base_blk, 0], sb2.at[0])
        for kk in range(_QG):
            pltpu.async_copy(xin.at[gb2.at[0, kk]], rows6.at[kk], sgat)

        def _chunk(k, carry):
            blk = k // 8
            sel = lax.rem(blk, 2)
            koff = lax.rem(k, 8)
            kq = k + _QG

            @pl.when(koff == 5)
            def _():
                nsel = lax.rem(blk + 1, 2)
                pltpu.make_async_copy(
                    gidx.at[h, base_blk, 0], gb2.at[nsel], sidxs).wait()
                pltpu.make_async_copy(
                    sidx.at[h, base_blk, 0], sb2.at[nsel], sidxs).wait()

            @pl.when(kq < nch_t)
            def _():
                @pl.when(kq >= _NB_ROWS)
                def _():
                    pltpu.make_async_copy(
                        rows6.at[0], acc.at[sb2.at[0, 0]], ssca).wait()
                qsel = lax.rem(kq // 8, 2)
                qoff = lax.rem(kq, 8)
                pltpu.async_copy(
                    xin.at[gb2.at[qsel, qoff]],
                    rows6.at[lax.rem(kq, _NB_ROWS)], sgat)

            @pl.when(koff == 2)
            def _():
                nb = jnp.minimum(blk + 1, BPT - 1)
                nsel = lax.rem(blk + 1, 2)
                pltpu.async_copy(gidx.at[h, base_blk + nb, 0],
                                 gb2.at[nsel], sidxs)
                pltpu.async_copy(sidx.at[h, base_blk + nb, 0],
                                 sb2.at[nsel], sidxs)

            pltpu.make_async_copy(
                xin.at[gb2.at[sel, koff]],
                rows6.at[lax.rem(k, _NB_ROWS)], sgat).wait()
            pltpu.async_copy(
                rows6.at[lax.rem(k, _NB_ROWS)],
                acc.at[sb2.at[sel, koff]], ssca, add=True)
            return carry

        lax.fori_loop(0, nch_t, _chunk, 0)
        for _ in range(_NB_ROWS):
            pltpu.make_async_copy(
                rows6.at[0], acc.at[sb2.at[0, 0]], ssca).wait()
        plsc.subcore_barrier()
        pltpu.sync_copy(
            acc.at[pl.ds(s * RPT, RPT)],
            out.at[pl.ds(h * NPAD + s * RPT, RPT)],
        )
        plsc.subcore_barrier()


@functools.lru_cache(maxsize=None)
def _sc_stage_fn():
    return pl.kernel(
        _sc_stage_body,
        out_type=jax.ShapeDtypeStruct((4 * NPAD, D), jnp.float32),
        mesh=_sc_mesh(),
        scratch_types=[
            pltpu.VMEM_SHARED((NPAD, D), jnp.float32),
            pltpu.VMEM((2, 8, CHS), jnp.int32),
            pltpu.VMEM((2, 8, CHS), jnp.int32),
            pltpu.VMEM((_NB_ROWS, CHS, D), jnp.float32),
            pltpu.SemaphoreType.DMA,
            pltpu.SemaphoreType.DMA,
            pltpu.SemaphoreType.DMA,
        ],
    )


def _sc_stage(xin_flat, gidx_abs, sidx_loc):
    return _sc_stage_fn()(xin_flat, gidx_abs, sidx_loc)


def _tc_scale_body(a_ref, bv_ref, o_ref):
    bv = bv_ref[0, 0, :]
    inv = jnp.where(bv > 0, 1.0 / bv, 0.0)
    o_ref[...] = a_ref[...] * inv[:, None]


def _tc_scale(a_flat, bv):
    return pl.pallas_call(
        _tc_scale_body,
        grid=(4,),
        in_specs=[
            pl.BlockSpec((NPAD, D), lambda i: (i, 0)),
            pl.BlockSpec((1, 1, NPAD), lambda i: (i, 0, 0)),
        ],
        out_specs=pl.BlockSpec((NPAD, D), lambda i: (i, 0)),
        out_shape=jax.ShapeDtypeStruct((4 * NPAD, D), jnp.float32),
    )(a_flat, bv.reshape(4, 1, NPAD))


def _tc_mid_body(c_ref, dv_ref, w1_ref, b1_ref, g_ref, be_ref, rm_ref, rv_ref,
                 w2_ref, o_ref):
    dv = dv_ref[0, 0, :]
    dinv = jnp.where(dv > 0, 1.0 / dv, 0.0)
    sg = g_ref[0, 0, :] / jnp.sqrt(rv_ref[0, 0, :] + 1e-5)
    t = be_ref[0, 0, :] - rm_ref[0, 0, :] * sg
    w1 = w1_ref[0]
    w2 = w2_ref[0]
    w1s = w1 * sg[:, None]
    weff = lax.dot_general(w1s, w2, (((0,), (1,)), ((), ())),
                           preferred_element_type=jnp.float32)
    bvec = (b1_ref[0, 0, :] * sg + t)[None, :]
    beff = lax.dot_general(bvec, w2, (((1,), (1,)), ((), ())),
                           preferred_element_type=jnp.float32)
    cd = c_ref[...] * dinv[:, None]
    o_ref[...] = lax.dot_general(cd, weff, (((1,), (0,)), ((), ())),
                                 preferred_element_type=jnp.float32) + beff


def _tc_mid(c_flat, dv, w1s_, b1s_, gs_, bes_, rms_, rvs_, w2s_):
    vec = lambda x: x.reshape(4, 1, 2 * D)
    return pl.pallas_call(
        _tc_mid_body,
        grid=(4,),
        in_specs=[
            pl.BlockSpec((NPAD, D), lambda i: (i, 0)),
            pl.BlockSpec((1, 1, NPAD), lambda i: (i, 0, 0)),
            pl.BlockSpec((1, 2 * D, D), lambda i: (i, 0, 0)),
            pl.BlockSpec((1, 1, 2 * D), lambda i: (i, 0, 0)),
            pl.BlockSpec((1, 1, 2 * D), lambda i: (i, 0, 0)),
            pl.BlockSpec((1, 1, 2 * D), lambda i: (i, 0, 0)),
            pl.BlockSpec((1, 1, 2 * D), lambda i: (i, 0, 0)),
            pl.BlockSpec((1, 1, 2 * D), lambda i: (i, 0, 0)),
            pl.BlockSpec((1, D, 2 * D), lambda i: (i, 0, 0)),
        ],
        out_specs=pl.BlockSpec((NPAD, D), lambda i: (i, 0)),
        out_shape=jax.ShapeDtypeStruct((4 * NPAD, D), jnp.float32),
    )(c_flat, dv.reshape(4, 1, NPAD), w1s_, vec(b1s_), vec(gs_), vec(bes_),
      vec(rms_), vec(rvs_), w2s_)


def _tc_final_body(c_ref, dv_ref, b2_ref, o_ref):
    dv = dv_ref[0, 0, :]
    dinv = jnp.where(dv > 0, 1.0 / dv, 0.0)
    o_ref[...] = c_ref[...] * dinv[:, None] + b2_ref[0, 0, :][None, :]


def _tc_final(c_flat, dv, b2s_):
    return pl.pallas_call(
        _tc_final_body,
        grid=(4,),
        in_specs=[
            pl.BlockSpec((NPAD, D), lambda i: (i, 0)),
            pl.BlockSpec((1, 1, NPAD), lambda i: (i, 0, 0)),
            pl.BlockSpec((1, 1, D), lambda i: (i, 0, 0)),
        ],
        out_specs=pl.BlockSpec((NPAD, D), lambda i: (i, 0)),
        out_shape=jax.ShapeDtypeStruct((4 * NPAD, D), jnp.float32),
    )(c_flat, dv.reshape(4, 1, NPAD), b2s_.reshape(4, 1, D))


_BR = 1024
_NBLK = NPAD // _BR


def _tc_loss_body(z1_ref, z2_ref, o_ref, sr, sc_, acc):
    i = pl.program_id(0)
    j = pl.program_id(1)

    @pl.when(jnp.logical_and(i == 0, j == 0))
    def _():
        acc[0] = 0.0
        acc[1] = 0.0
        acc[2] = 0.0

    z1 = z1_ref[...]
    z2 = z2_ref[...]
    n1 = z1 / jnp.maximum(
        jnp.sqrt(jnp.sum(z1 * z1, axis=1, keepdims=True)), 1e-12)
    n2 = z2 / jnp.maximum(
        jnp.sqrt(jnp.sum(z2 * z2, axis=1, keepdims=True)), 1e-12)
    s_blk = lax.dot_general(n1.astype(jnp.bfloat16), n2.astype(jnp.bfloat16),
                            (((1,), (1,)), ((), ())),
                            preferred_element_type=jnp.float32) * (1.0 / 0.7)
    ri = lax.broadcasted_iota(jnp.int32, (_BR, _BR), 0)
    ci = lax.broadcasted_iota(jnp.int32, (_BR, _BR), 1)
    valid = jnp.logical_and(i * _BR + ri < N, j * _BR + ci < N)
    p_blk = jnp.where(valid, jnp.exp(s_blk), 0.0)

    @pl.when(j == 0)
    def _():
        sr[...] = jnp.zeros_like(sr)

    sr[...] = sr[...] + jnp.sum(p_blk, axis=1, keepdims=True)

    @pl.when(i == 0)
    def _():
        sc_[0, pl.ds(j * _BR, _BR)] = jnp.zeros((_BR,), jnp.float32)

    cur = sc_[0, pl.ds(j * _BR, _BR)]
    sc_[0, pl.ds(j * _BR, _BR)] = cur + jnp.sum(p_blk, axis=0)

    @pl.when(i == j)
    def _():
        dmask = jnp.logical_and(ri == ci, valid)
        acc[2] = acc[2] + jnp.sum(jnp.where(dmask, s_blk, 0.0))

    @pl.when(j == _NBLK - 1)
    def _():
        rv_ = lax.broadcasted_iota(jnp.int32, (_BR, 1), 0) + i * _BR < N
        acc[0] = acc[0] + jnp.sum(jnp.where(rv_, jnp.log(sr[...]), 0.0))

    @pl.when(jnp.logical_and(i == _NBLK - 1, j == _NBLK - 1))
    def _():
        cv_ = lax.broadcasted_iota(jnp.int32, (1, NPAD), 1) < N
        acc[1] = jnp.sum(jnp.where(cv_, jnp.log(sc_[...]), 0.0))
        val = 0.5 * ((acc[0] - acc[2]) + (acc[1] - acc[2])) / N
        o_ref[...] = jnp.full((1, 1), val, jnp.float32)


def _tc_loss(z1_pad, z2_pad):
    out = pl.pallas_call(
        _tc_loss_body,
        grid=(_NBLK, _NBLK),
        in_specs=[
            pl.BlockSpec((_BR, D), lambda i, j: (i, 0)),
            pl.BlockSpec((_BR, D), lambda i, j: (j, 0)),
        ],
        out_specs=pl.BlockSpec((1, 1), lambda i, j: (0, 0)),
        out_shape=jax.ShapeDtypeStruct((1, 1), jnp.float32),
        scratch_shapes=[
            pltpu.VMEM((_BR, 1), jnp.float32),
            pltpu.VMEM((1, NPAD), jnp.float32),
            pltpu.SMEM((3,), jnp.float32),
        ],
    )(z1_pad, z2_pad)
    return out[0, 0]


def kernel(x_protein, hyperedge_protein_index, x_meta, hyperedge_meta_index,
           pc1_w1, pc1_b1, pc1_g, pc1_be, pc1_rm, pc1_rv, pc1_w2, pc1_b2,
           pc2_w1, pc2_b1, pc2_g, pc2_be, pc2_rm, pc2_rv, pc2_w2, pc2_b2,
           mc1_w1, mc1_b1, mc1_g, mc1_be, mc1_rm, mc1_rv, mc1_w2, mc1_b2,
           mc2_w1, mc2_b1, mc2_g, mc2_be, mc2_rm, mc2_rv, mc2_w2, mc2_b2,
           p12, m13):
    hp = hyperedge_protein_index
    hm = hyperedge_meta_index
    src4 = jnp.stack([hp[0, 0], hp[1, 0], hm[0, 0], hm[1, 0]])
    dst4 = jnp.stack([hp[0, 1], hp[1, 1], hm[0, 1], hm[1, 1]])
    off = (jnp.arange(4, dtype=jnp.int32) * NPAD)[:, None]
    e_pad = jnp.arange(EP - E, dtype=jnp.int32)
    gpad = (e_pad % N)[None, :] + off
    spad = jnp.broadcast_to(N + (e_pad % (NPAD - N)), (4, EP - E)).astype(jnp.int32)
    to5d = lambda a: a.reshape(4, NBLK8, 1, 8, CHS)
    src_abs5 = to5d(jnp.concatenate([src4 + off, gpad], axis=1))
    dst_abs5 = to5d(jnp.concatenate([dst4 + off, gpad], axis=1))
    src_loc5 = to5d(jnp.concatenate([src4, spad], axis=1))
    dst_loc5 = to5d(jnp.concatenate([dst4, spad], axis=1))
    ones_ch = jnp.ones((CHS,), jnp.float32)

    w1s = jnp.stack([pc1_w1, pc2_w1, mc1_w1, mc2_w1])
    b1s = jnp.stack([pc1_b1, pc2_b1, mc1_b1, mc2_b1])
    gs = jnp.stack([pc1_g, pc2_g, mc1_g, mc2_g])
    bes = jnp.stack([pc1_be, pc2_be, mc1_be, mc2_be])
    rms = jnp.stack([pc1_rm, pc2_rm, mc1_rm, mc2_rm])
    rvs = jnp.stack([pc1_rv, pc2_rv, mc1_rv, mc2_rv])
    w2s = jnp.stack([pc1_w2, pc2_w2, mc1_w2, mc2_w2])
    b2s = jnp.stack([pc1_b2, pc2_b2, mc1_b2, mc2_b2])

    pad = jnp.zeros((NPAD - N, D), jnp.float32)
    x4 = jnp.concatenate([
        x_protein, pad, x_protein, pad, x_meta, pad, x_meta, pad
    ]).reshape(4 * NPAD, D)

    dv, bv = _sc_hist(src_loc5, dst_loc5, ones_ch)

    a1 = _sc_stage(x4, src_abs5, dst_loc5)
    b1_ = _tc_scale(a1, bv)
    c1 = _sc_stage(b1_, dst_abs5, src_loc5)
    z = _tc_mid(c1, dv, w1s, b1s, gs, bes, rms, rvs, w2s)

    a2 = _sc_stage(z, src_abs5, dst_loc5)
    b2_ = _tc_scale(a2, bv)
    c2 = _sc_stage(b2_, dst_abs5, src_loc5)
    feats_pad = _tc_final(c2, dv, b2s).reshape(4, NPAD, D)
    feats = feats_pad[:, :N]

    loss_p = _tc_loss(feats_pad[0], feats_pad[1])
    loss_m = _tc_loss(feats_pad[2], feats_pad[3])
    loss = jnp.exp(-p12) * loss_p + p12 + jnp.exp(-m13) * loss_m + m13

    protein = jnp.stack((feats[1], feats[0]))[None]
    meta = jnp.stack((feats[3], feats[2]))[None]
    return (protein, meta, x_protein[None, None], x_meta[None, None], loss)
